# baseline (speedup 1.0000x reference)
import functools

import jax
import jax.numpy as jnp
from jax import lax
from jax.experimental import pallas as pl
from jax.experimental.pallas import tpu as pltpu
from jax.experimental.pallas import tpu_sc as plsc

_B, _T, _H, _V = 1024, 200, 64, 100000
_NST, _NKC, _NKR, _NOBS = 3, 2, 4, 3
_NW = 32
_G = 64
_BT = _B * _T
_PW = _BT // _NW
_NBLK = _PW // _G
_SPW = (_B * _NST) // _NW
_KROW = _H * (_NKR + _NKC)
_OROW = _H * _NOBS
_HC = _H // 16


@functools.partial(
    pl.kernel,
    out_type=(
        jax.ShapeDtypeStruct((_B * _NST, _H), jnp.float32),
        jax.ShapeDtypeStruct((_BT * _KROW,), jnp.float32),
        jax.ShapeDtypeStruct((_BT * _OROW,), jnp.float32),
    ),
    mesh=plsc.VectorSubcoreMesh(core_axis_name="c", subcore_axis_name="s"),
    compiler_params=pltpu.CompilerParams(needs_layout_passes=False,
                                         use_tc_tiling_on_sc=False),
    scratch_types=[
        pltpu.VMEM((_NKR, _H), jnp.float32),
        pltpu.VMEM((_NKR, _H), jnp.float32),
        pltpu.VMEM((_NOBS, _H), jnp.float32),
        pltpu.VMEM((_NOBS, _H), jnp.float32),
        pltpu.VMEM((_G * 8 + 16,), jnp.float32),
        pltpu.VMEM((_G * _NKC,), jnp.int32),
        pltpu.VMEM((_G * _NKC, _H), jnp.float32),
        pltpu.VMEM((_SPW,), jnp.int32),
        pltpu.VMEM((_SPW, _H), jnp.float32),
        pltpu.VMEM((_G * _KROW,), jnp.float32),
        pltpu.VMEM((_G * _OROW,), jnp.float32),
        pltpu.SemaphoreType.DMA,
    ],
)
def _sc_embed(scal_hbm, kcix_hbm, stix_hbm, ekc_hbm, est_hbm,
              wkr_hbm, bkr_hbm, wob_hbm, bob_hbm,
              st_out, k_out, o_out,
              wkr_v, bkr_v, wob_v, bob_v,
              scal_v, kcix_v, krows_v, stix_v, srows_v,
              krow_v, orow_v, sem):
    wid = lax.axis_index("s") * 2 + lax.axis_index("c")

    pltpu.sync_copy(wkr_hbm, wkr_v)
    pltpu.sync_copy(bkr_hbm, bkr_v)
    pltpu.sync_copy(wob_hbm, wob_v)
    pltpu.sync_copy(bob_hbm, bob_v)

    sbase = wid * _SPW
    pltpu.sync_copy(stix_hbm.at[pl.ds(sbase, _SPW)], stix_v)
    pltpu.async_copy(est_hbm.at[stix_v], srows_v, sem).wait()
    pltpu.sync_copy(srows_v, st_out.at[pl.ds(sbase, _SPW)])

    pos6 = jnp.arange(16, dtype=jnp.int32) * 6
    pos3 = jnp.arange(16, dtype=jnp.int32) * 3
    wk = [[wkr_v[j, pl.ds(c * 16, 16)] for c in range(_HC)] for j in range(_NKR)]
    wo = [[wob_v[j, pl.ds(c * 16, 16)] for c in range(_HC)] for j in range(_NOBS)]

    def block(blk, carry):
        base = wid * _PW + blk * _G
        pltpu.sync_copy(scal_hbm.at[pl.ds(base * 8, _G * 8)],
                        scal_v.at[pl.ds(0, _G * 8)])
        pltpu.sync_copy(kcix_hbm.at[pl.ds(base * _NKC, _G * _NKC)], kcix_v)
        pltpu.async_copy(ekc_hbm.at[kcix_v], krows_v, sem).wait()

        def pair(g, c2):
            kb = g * _KROW
            obb = g * _OROW
            sv = scal_v[pl.ds(g * 8, 16)]
            for j in range(_NKR):
                s = sv[j]
                for c in range(_HC):
                    val = s * wk[j][c] + bkr_v[j, pl.ds(c * 16, 16)]
                    plsc.store_scatter(krow_v, [pos6 + (kb + 96 * c + j)], val)
            for i in range(_NKC):
                for c in range(_HC):
                    val = krows_v[g * _NKC + i, pl.ds(c * 16, 16)]
                    plsc.store_scatter(krow_v, [pos6 + (kb + 96 * c + _NKR + i)], val)
            for j in range(_NOBS):
                s = sv[_NKR + j]
                for c in range(_HC):
                    val = s * wo[j][c] + bob_v[j, pl.ds(c * 16, 16)]
                    plsc.store_scatter(orow_v, [pos3 + (obb + 48 * c + j)], val)
            return c2

        lax.fori_loop(0, _G, pair, 0)
        pltpu.sync_copy(krow_v, k_out.at[pl.ds(base * _KROW, _G * _KROW)])
        pltpu.sync_copy(orow_v, o_out.at[pl.ds(base * _OROW, _G * _OROW)])
        return carry

    lax.fori_loop(0, _NBLK, block, 0)


def kernel(static, known_real, known_categorical, observed,
           E_static, E_kc, W_kr, b_kr, W_obs, b_obs):
    scal = jnp.concatenate(
        [known_real.reshape(_BT, _NKR), observed.reshape(_BT, _NOBS),
         jnp.zeros((_BT, 1), jnp.float32)], axis=1).reshape(_BT * 8)
    kcix = (known_categorical.astype(jnp.int32)
            + jnp.arange(_NKC, dtype=jnp.int32) * _V).reshape(_BT * _NKC)
    stix = (static.astype(jnp.int32)
            + jnp.arange(_NST, dtype=jnp.int32) * _V).reshape(_B * _NST)
    st_out, k_out, o_out = _sc_embed(
        scal, kcix, stix,
        E_kc.reshape(_NKC * _V, _H), E_static.reshape(_NST * _V, _H),
        W_kr, b_kr, W_obs, b_obs)
    static_emb = st_out.reshape(_B, _NST, _H)
    known = k_out.reshape(_B, _T, _H, _NKR + _NKC)
    obs = o_out.reshape(_B, _T, _H, _NOBS)
    return (static_emb, known, obs)


# speedup vs baseline: 9.1081x; 9.1081x over previous
import functools

import jax
import jax.numpy as jnp
from jax import lax
from jax.experimental import pallas as pl
from jax.experimental.pallas import tpu as pltpu
from jax.experimental.pallas import tpu_sc as plsc

_B, _T, _H, _V = 1024, 200, 64, 100000
_NST, _NKC, _NKR, _NOBS = 3, 2, 4, 3
_NW = 32
_VB = 128
_VP = 100096
_NBLK = _VP // _VB
_K1_ITERS = -(-_NBLK // _NW)
_NUNIT = _T * 8
_UPW = _NUNIT // _NW

_MESH = plsc.VectorSubcoreMesh(core_axis_name="c", subcore_axis_name="s")
_CPARAMS = pltpu.CompilerParams(needs_layout_passes=False)


@functools.partial(
    pl.kernel,
    out_type=(
        jax.ShapeDtypeStruct((_VP, 2 * _H), jnp.float32),
        jax.ShapeDtypeStruct((_VP, 4 * _H), jnp.float32),
    ),
    mesh=_MESH,
    compiler_params=_CPARAMS,
    scratch_types=[
        pltpu.VMEM((_H, _VB), jnp.float32),
        pltpu.VMEM((_VB, 2 * _H), jnp.float32),
        pltpu.VMEM((_VB, 4 * _H), jnp.float32),
    ],
)
def _sc_reformat(ekcv, estv, ekc2, est2, tin, tkc, tst):
    wid = lax.axis_index("s") * 2 + lax.axis_index("c")
    lane = lax.iota(jnp.int32, 16)
    rowv = [lane + c * 16 for c in range(_VB // 16)]

    def block(it, carry):
        blk = it * _NW + wid

        @pl.when(blk < _NBLK)
        def _():
            v0 = pl.multiple_of(blk * _VB, _VB)
            for f in range(_NKC):
                pltpu.sync_copy(ekcv.at[f, :, pl.ds(v0, _VB)], tin)

                def hbody(h, c2, f=f):
                    col = jnp.full((16,), f * _H + h, jnp.int32)
                    for c in range(_VB // 16):
                        val = tin[h, pl.ds(c * 16, 16)]
                        plsc.store_scatter(tkc, [rowv[c], col], val)
                    return c2

                lax.fori_loop(0, _H, hbody, 0)
            pltpu.sync_copy(tkc, ekc2.at[pl.ds(v0, _VB)])
            for f in range(_NST):
                pltpu.sync_copy(estv.at[f, :, pl.ds(v0, _VB)], tin)

                def hbody2(h, c2, f=f):
                    col = jnp.full((16,), f * _H + h, jnp.int32)
                    for c in range(_VB // 16):
                        val = tin[h, pl.ds(c * 16, 16)]
                        plsc.store_scatter(tst, [rowv[c], col], val)
                    return c2

                lax.fori_loop(0, _H, hbody2, 0)
            pltpu.sync_copy(tst, est2.at[pl.ds(v0, _VB)])

        return carry

    lax.fori_loop(0, _K1_ITERS, block, 0)


@functools.partial(
    pl.kernel,
    out_type=(
        jax.ShapeDtypeStruct((_T * 6 * 8, 8, 8, 128), jnp.float32),
        jax.ShapeDtypeStruct((_T * 3 * 8, 8, 8, 128), jnp.float32),
        jax.ShapeDtypeStruct((_NST * 8, 8, 8, 128), jnp.float32),
    ),
    mesh=_MESH,
    compiler_params=_CPARAMS,
    scratch_types=[
        pltpu.VMEM((8, 128), jnp.float32),
        pltpu.VMEM((128,), jnp.int32),
        pltpu.VMEM((128,), jnp.int32),
        pltpu.VMEM((128, 4 * _H), jnp.float32),
        pltpu.VMEM((128, 2 * _H), jnp.float32),
        pltpu.VMEM((48, 8, 128), jnp.float32),
        pltpu.VMEM((256,), jnp.float32),
        pltpu.VMEM((256,), jnp.float32),
        pltpu.VMEM((192,), jnp.float32),
        pltpu.VMEM((192,), jnp.float32),
        pltpu.SemaphoreType.DMA,
    ],
)
def _sc_main(scal2, kcidx, statx, ekc2, est2, wkr1, bkr1, wob1, bob1,
             ko, oo, so,
             sv, i0, i1, g0, g1, slab, wkr_v, bkr_v, wob_v, bob_v, sem):
    wid = lax.axis_index("s") * 2 + lax.axis_index("c")
    lane = lax.iota(jnp.int32, 16)
    rowbase = [lane + c * 16 for c in range(8)]

    pltpu.sync_copy(wkr1, wkr_v)
    pltpu.sync_copy(bkr1, bkr_v)
    pltpu.sync_copy(wob1, wob_v)
    pltpu.sync_copy(bob1, bob_v)

    def dense_slab(j, wref, bref, woff, row0):
        srow = [sv[j, pl.ds(c * 16, 16)] for c in range(8)]

        def hbody(h, c2):
            hq = h // 8
            hr = h % 8
            hsplat = jnp.full((16,), woff + h, jnp.int32)
            wv = plsc.load_gather(wref, [hsplat])
            bv = plsc.load_gather(bref, [hsplat])
            for c in range(8):
                slab[row0 + hq, hr, pl.ds(c * 16, 16)] = srow[c] * wv + bv
            return c2

        lax.fori_loop(0, _H, hbody, 0)

    def kc_slab(g, coff, row0):
        def hbody(h, c2):
            hq = h // 8
            hr = h % 8
            col = jnp.full((16,), coff + h, jnp.int32)
            for c in range(8):
                val = plsc.load_gather(g, [rowbase[c], col])
                slab[row0 + hq, hr, pl.ds(c * 16, 16)] = val
            return c2

        lax.fori_loop(0, _H, hbody, 0)

    def unit(ui, carry):
        u = wid * _UPW + ui
        t = u // 8
        bb = u % 8
        b0 = pl.multiple_of(bb * 128, 128)
        pltpu.sync_copy(scal2.at[t, :, pl.ds(b0, 128)], sv)
        pltpu.sync_copy(kcidx.at[t, 0, pl.ds(b0, 128)], i0)
        pltpu.sync_copy(kcidx.at[t, 1, pl.ds(b0, 128)], i1)
        pltpu.async_copy(ekc2.at[i0], g1, sem).wait()
        for j in range(_NKR):
            dense_slab(j, wkr_v, bkr_v, j * _H, j * 8)
        kc_slab(g1, 0, 4 * 8)
        pltpu.async_copy(ekc2.at[i1], g1, sem).wait()
        kc_slab(g1, _H, 5 * 8)
        pltpu.sync_copy(slab, ko.at[pl.ds(t * 48, 48), bb])
        for j in range(_NOBS):
            dense_slab(_NKR + j, wob_v, bob_v, j * _H, j * 8)
        pltpu.sync_copy(slab.at[pl.ds(0, 24)], oo.at[pl.ds(t * 24, 24), bb])
        return carry

    lax.fori_loop(0, _UPW, unit, 0)

    @pl.when(wid < _NST * 8)
    def _():
        f = wid // 8
        bb = wid % 8
        b0 = pl.multiple_of(bb * 128, 128)
        pltpu.sync_copy(statx.at[f, pl.ds(b0, 128)], i0)
        pltpu.async_copy(est2.at[i0], g0, sem).wait()

        def hbody(h, c2):
            hq = h // 8
            hr = h % 8
            col = jnp.full((16,), f * _H + h, jnp.int32)
            for c in range(8):
                val = plsc.load_gather(g0, [rowbase[c], col])
                slab[hq, hr, pl.ds(c * 16, 16)] = val
            return c2

        lax.fori_loop(0, _H, hbody, 0)
        pltpu.sync_copy(slab.at[pl.ds(0, 8)], so.at[pl.ds(f * 8, 8), bb])


def kernel(static, known_real, known_categorical, observed,
           E_static, E_kc, W_kr, b_kr, W_obs, b_obs):
    ekcv = jnp.swapaxes(E_kc, 1, 2)
    estv = jnp.swapaxes(E_static, 1, 2)
    ekc2, est2 = _sc_reformat(ekcv, estv)
    scal2 = jnp.transpose(
        jnp.concatenate(
            [known_real, observed, jnp.zeros((_B, _T, 1), jnp.float32)],
            axis=-1),
        (1, 2, 0))
    kcidx = jnp.transpose(known_categorical.astype(jnp.int32), (1, 2, 0))
    statx = jnp.transpose(static.astype(jnp.int32), (1, 0))
    ko, oo, so = _sc_main(scal2, kcidx, statx, ekc2, est2,
                          W_kr.reshape(-1), b_kr.reshape(-1),
                          W_obs.reshape(-1), b_obs.reshape(-1))
    known = (ko.reshape(_T, 6, 8, 8, 8, 128)
             .transpose(3, 5, 0, 2, 4, 1)
             .reshape(_B, _T, _H, 6))
    obs = (oo.reshape(_T, 3, 8, 8, 8, 128)
           .transpose(3, 5, 0, 2, 4, 1)
           .reshape(_B, _T, _H, 3))
    static_emb = (so.reshape(_NST, 8, 8, 8, 128)
                  .transpose(2, 4, 0, 1, 3)
                  .reshape(_B, _NST, _H))
    return (static_emb, known, obs)


# speedup vs baseline: 10.8471x; 1.1909x over previous
import functools

import jax
import jax.numpy as jnp
from jax import lax
from jax.experimental import pallas as pl
from jax.experimental.pallas import tpu as pltpu
from jax.experimental.pallas import tpu_sc as plsc

_B, _T, _H, _V = 1024, 200, 64, 100000
_NST, _NKC, _NKR, _NOBS = 3, 2, 4, 3
_NW = 32
_VB = 128
_VP = 100096
_NBLK = _VP // _VB
_K1_ITERS = -(-_NBLK // _NW)
_NUNIT = _T * 8
_UPW = _NUNIT // _NW

_MESH = plsc.VectorSubcoreMesh(core_axis_name="c", subcore_axis_name="s")
_CPARAMS = pltpu.CompilerParams(needs_layout_passes=False)


@functools.partial(
    pl.kernel,
    out_type=(
        jax.ShapeDtypeStruct((_VP, 2 * _H), jnp.float32),
        jax.ShapeDtypeStruct((_VP, 2 * _H), jnp.float32),
        jax.ShapeDtypeStruct((_VP, 2 * _H), jnp.float32),
    ),
    mesh=_MESH,
    compiler_params=_CPARAMS,
    scratch_types=[
        pltpu.VMEM((5, _H, _VB), jnp.float32),
        pltpu.VMEM((_VB, 2 * _H), jnp.float32),
        pltpu.VMEM((_VB, 2 * _H), jnp.float32),
        pltpu.VMEM((_VB, 2 * _H), jnp.float32),
        pltpu.SemaphoreType.DMA,
        pltpu.SemaphoreType.DMA,
        pltpu.SemaphoreType.DMA,
        pltpu.SemaphoreType.DMA,
        pltpu.SemaphoreType.DMA,
        pltpu.SemaphoreType.DMA,
    ],
)
def _sc_reformat(ekcv, estv, ekc2, esta, estb,
                 tin, tkc, tsta, tstb, si0, si1, si2, si3, si4, sem_out):
    wid = lax.axis_index("s") * 2 + lax.axis_index("c")
    lane = lax.iota(jnp.int32, 16)
    rowv = [lane + c * 16 for c in range(_VB // 16)]
    sin = [si0, si1, si2, si3, si4]

    def transpose_field(slot, dst, coff):
        pltpu.make_async_copy(ekcv.at[0, :, pl.ds(0, _VB)],
                              tin.at[slot], sin[slot]).wait()

        def hbody(h, c2):
            col = jnp.full((16,), coff + h, jnp.int32)
            for c in range(_VB // 16):
                val = tin[slot, h, pl.ds(c * 16, 16)]
                plsc.store_scatter(dst, [rowv[c], col], val)
            return c2

        lax.fori_loop(0, _H, hbody, 0)

    def block(it, carry):
        blk = it * _NW + wid

        @pl.when(blk < _NBLK)
        def _():
            v0 = pl.multiple_of(blk * _VB, _VB)
            for f in range(_NKC):
                pltpu.async_copy(ekcv.at[f, :, pl.ds(v0, _VB)],
                                 tin.at[f], sin[f])
            for f in range(_NST):
                pltpu.async_copy(estv.at[f, :, pl.ds(v0, _VB)],
                                 tin.at[_NKC + f], sin[_NKC + f])

            @pl.when(it > 0)
            def _():
                pltpu.make_async_copy(tkc, ekc2.at[pl.ds(0, _VB)],
                                      sem_out).wait()
                pltpu.make_async_copy(tsta, esta.at[pl.ds(0, _VB)],
                                      sem_out).wait()
                pltpu.make_async_copy(tstb, estb.at[pl.ds(0, _VB)],
                                      sem_out).wait()

            transpose_field(0, tkc, 0)
            transpose_field(1, tkc, _H)
            pltpu.async_copy(tkc, ekc2.at[pl.ds(v0, _VB)], sem_out)
            transpose_field(2, tsta, 0)
            transpose_field(3, tsta, _H)
            pltpu.async_copy(tsta, esta.at[pl.ds(v0, _VB)], sem_out)
            transpose_field(4, tstb, 0)
            pltpu.async_copy(tstb, estb.at[pl.ds(v0, _VB)], sem_out)

        return carry

    lax.fori_loop(0, _K1_ITERS, block, 0)
    pltpu.make_async_copy(tkc, ekc2.at[pl.ds(0, _VB)], sem_out).wait()
    pltpu.make_async_copy(tsta, esta.at[pl.ds(0, _VB)], sem_out).wait()
    pltpu.make_async_copy(tstb, estb.at[pl.ds(0, _VB)], sem_out).wait()


@functools.partial(
    pl.kernel,
    out_type=(
        jax.ShapeDtypeStruct((_T * 6 * 8, 8, 8, 128), jnp.float32),
        jax.ShapeDtypeStruct((_T * 3 * 8, 8, 8, 128), jnp.float32),
        jax.ShapeDtypeStruct((_NST * 8, 8, 8, 128), jnp.float32),
    ),
    mesh=_MESH,
    compiler_params=_CPARAMS,
    scratch_types=[
        pltpu.VMEM((8, 128), jnp.float32),
        pltpu.VMEM((128,), jnp.int32),
        pltpu.VMEM((128,), jnp.int32),
        pltpu.VMEM((128, 2 * _H), jnp.float32),
        pltpu.VMEM((128, 2 * _H), jnp.float32),
        pltpu.VMEM((48, 8, 128), jnp.float32),
        pltpu.VMEM((24, 8, 128), jnp.float32),
        pltpu.VMEM((256,), jnp.float32),
        pltpu.VMEM((256,), jnp.float32),
        pltpu.VMEM((192,), jnp.float32),
        pltpu.VMEM((192,), jnp.float32),
        pltpu.SemaphoreType.DMA,
        pltpu.SemaphoreType.DMA,
        pltpu.SemaphoreType.DMA,
        pltpu.SemaphoreType.DMA,
        pltpu.SemaphoreType.DMA,
    ],
)
def _sc_main(scal2, kcidx, statx, ekc2, esta, estb, wkr1, bkr1, wob1, bob1,
             ko, oo, so,
             sv, i0, i1, ga, gb, kslab, oslab, wkr_v, bkr_v, wob_v, bob_v,
             sem_sv, sem_ii, sem_g, sem_ko, sem_oo):
    wid = lax.axis_index("s") * 2 + lax.axis_index("c")
    lane = lax.iota(jnp.int32, 16)
    rowbase = [lane + c * 16 for c in range(8)]

    pltpu.sync_copy(wkr1, wkr_v)
    pltpu.sync_copy(bkr1, bkr_v)
    pltpu.sync_copy(wob1, wob_v)
    pltpu.sync_copy(bob1, bob_v)

    def fire_in(u):
        t = u // 8
        b0 = pl.multiple_of((u % 8) * 128, 128)
        pltpu.async_copy(scal2.at[t, :, pl.ds(b0, 128)], sv, sem_sv)
        pltpu.async_copy(kcidx.at[t, 0, pl.ds(b0, 128)], i0, sem_ii)
        pltpu.async_copy(kcidx.at[t, 1, pl.ds(b0, 128)], i1, sem_ii)

    def dense_slab(slab, j, wref, bref, woff, row0):
        srow = [sv[j, pl.ds(c * 16, 16)] for c in range(8)]

        def hbody(h, c2):
            hq = h // 8
            hr = h % 8
            hsplat = jnp.full((16,), woff + h, jnp.int32)
            wv = plsc.load_gather(wref, [hsplat])
            bv = plsc.load_gather(bref, [hsplat])
            for c in range(8):
                slab[row0 + hq, hr, pl.ds(c * 16, 16)] = srow[c] * wv + bv
            return c2

        lax.fori_loop(0, _H, hbody, 0)

    def kc_slab(slab, g, coff, row0):
        def hbody(h, c2):
            hq = h // 8
            hr = h % 8
            col = coff + h
            colv = jnp.full((16,), col, jnp.int32)
            for c in range(8):
                val = plsc.load_gather(g, [rowbase[c], colv])
                slab[row0 + hq, hr, pl.ds(c * 16, 16)] = val
            return c2

        lax.fori_loop(0, _H, hbody, 0)

    fire_in(wid * _UPW)

    def unit(ui, carry):
        u = wid * _UPW + ui
        t = u // 8
        bb = u % 8
        pltpu.make_async_copy(scal2.at[0, :, pl.ds(0, 128)], sv, sem_sv).wait()
        pltpu.make_async_copy(kcidx.at[0, 0, pl.ds(0, 128)], i0, sem_ii).wait()
        pltpu.make_async_copy(kcidx.at[0, 1, pl.ds(0, 128)], i1, sem_ii).wait()
        pltpu.async_copy(ekc2.at[i0], ga, sem_g)
        pltpu.async_copy(ekc2.at[i1], gb, sem_g)

        @pl.when(ui > 0)
        def _():
            pltpu.make_async_copy(kslab, ko.at[pl.ds(0, 48), 0], sem_ko).wait()

        for j in range(_NKR):
            dense_slab(kslab, j, wkr_v, bkr_v, j * _H, j * 8)

        @pl.when(ui > 0)
        def _():
            pltpu.make_async_copy(oslab, oo.at[pl.ds(0, 24), 0], sem_oo).wait()

        for j in range(_NOBS):
            dense_slab(oslab, _NKR + j, wob_v, bob_v, j * _H, j * 8)

        pltpu.make_async_copy(ekc2.at[pl.ds(0, 128)], ga, sem_g).wait()
        pltpu.make_async_copy(ekc2.at[pl.ds(0, 128)], gb, sem_g).wait()
        kc_slab(kslab, ga, 0, 4 * 8)
        kc_slab(kslab, gb, _H, 5 * 8)

        @pl.when(ui + 1 < _UPW)
        def _():
            fire_in(u + 1)

        pltpu.async_copy(kslab, ko.at[pl.ds(t * 48, 48), bb], sem_ko)
        pltpu.async_copy(oslab, oo.at[pl.ds(t * 24, 24), bb], sem_oo)
        return carry

    lax.fori_loop(0, _UPW, unit, 0)
    pltpu.make_async_copy(kslab, ko.at[pl.ds(0, 48), 0], sem_ko).wait()
    pltpu.make_async_copy(oslab, oo.at[pl.ds(0, 24), 0], sem_oo).wait()

    @pl.when(wid < _NST * 8)
    def _():
        f = wid // 8
        bb = wid % 8
        b0 = pl.multiple_of(bb * 128, 128)
        pltpu.sync_copy(statx.at[f, pl.ds(b0, 128)], i0)
        for ff in range(2):
            @pl.when(f == ff)
            def _():
                pltpu.async_copy(esta.at[i0], ga, sem_g).wait()

        @pl.when(f == 2)
        def _():
            pltpu.async_copy(estb.at[i0], ga, sem_g).wait()

        coff = (f % 2) * _H

        def hbody(h, c2):
            hq = h // 8
            hr = h % 8
            colv = jnp.full((16,), coff + h, jnp.int32)
            for c in range(8):
                val = plsc.load_gather(ga, [rowbase[c], colv])
                oslab[hq, hr, pl.ds(c * 16, 16)] = val
            return c2

        lax.fori_loop(0, _H, hbody, 0)
        pltpu.sync_copy(oslab.at[pl.ds(0, 8)], so.at[pl.ds(f * 8, 8), bb])


def kernel(static, known_real, known_categorical, observed,
           E_static, E_kc, W_kr, b_kr, W_obs, b_obs):
    ekcv = jnp.swapaxes(E_kc, 1, 2)
    estv = jnp.swapaxes(E_static, 1, 2)
    ekc2, esta, estb = _sc_reformat(ekcv, estv)
    scal2 = jnp.transpose(
        jnp.concatenate(
            [known_real, observed, jnp.zeros((_B, _T, 1), jnp.float32)],
            axis=-1),
        (1, 2, 0))
    kcidx = jnp.transpose(known_categorical.astype(jnp.int32), (1, 2, 0))
    statx = jnp.transpose(static.astype(jnp.int32), (1, 0))
    ko, oo, so = _sc_main(scal2, kcidx, statx, ekc2, esta, estb,
                          W_kr.reshape(-1), b_kr.reshape(-1),
                          W_obs.reshape(-1), b_obs.reshape(-1))
    known = (ko.reshape(_T, 6, 8, 8, 8, 128)
             .transpose(3, 5, 0, 2, 4, 1)
             .reshape(_B, _T, _H, 6))
    obs = (oo.reshape(_T, 3, 8, 8, 8, 128)
           .transpose(3, 5, 0, 2, 4, 1)
           .reshape(_B, _T, _H, 3))
    static_emb = (so.reshape(_NST, 8, 8, 8, 128)
                  .transpose(2, 4, 0, 1, 3)
                  .reshape(_B, _NST, _H))
    return (static_emb, known, obs)
